# Initial kernel scaffold; baseline (speedup 1.0000x reference)
import functools

import jax
import jax.numpy as jnp
from jax import lax
from jax.experimental import pallas as pl
from jax.experimental.pallas import tpu as pltpu
from jax.experimental.pallas import tpu_sc as plsc

DIM = 32
BATCH = 4096
HIST = 200
TOTAL = BATCH * HIST
NUM_CORES = 2
NUM_SUBCORES = 16
NW = NUM_CORES * NUM_SUBCORES
PER_W = TOTAL // NW
CHUNK = 1280
NCHUNK = PER_W // CHUNK

_mesh = plsc.VectorSubcoreMesh(core_axis_name="c", subcore_axis_name="s")


@functools.partial(
    pl.kernel,
    mesh=_mesh,
    out_type=jax.ShapeDtypeStruct((TOTAL, DIM), jnp.float32),
    scratch_types=[
        pltpu.VMEM((CHUNK,), jnp.int32),
        pltpu.VMEM((CHUNK, DIM), jnp.float32),
        pltpu.SemaphoreType.DMA,
    ],
)
def _embed(tokens_hbm, table_hbm, out_hbm, idx_v, rows_v, sem):
    wid = lax.axis_index("s") * NUM_CORES + lax.axis_index("c")
    base = wid * PER_W

    def body(c, carry):
        b0 = base + c * CHUNK
        pltpu.sync_copy(tokens_hbm.at[pl.ds(b0, CHUNK)], idx_v)
        pltpu.async_copy(table_hbm.at[idx_v], rows_v, sem).wait()
        pltpu.sync_copy(rows_v, out_hbm.at[pl.ds(b0, CHUNK)])
        return carry

    lax.fori_loop(0, NCHUNK, body, 0)


def kernel(tokens, table):
    flat = tokens.reshape(TOTAL).astype(jnp.int32)
    out = _embed(flat, table)
    return out.reshape(BATCH, HIST, DIM)


# speedup vs baseline: 1.4697x; 1.4697x over previous
import functools

import jax
import jax.numpy as jnp
from jax import lax
from jax.experimental import pallas as pl
from jax.experimental.pallas import tpu as pltpu
from jax.experimental.pallas import tpu_sc as plsc

DIM = 32
BATCH = 4096
HIST = 200
TOTAL = BATCH * HIST
NUM_CORES = 2
NUM_SUBCORES = 16
NW = NUM_CORES * NUM_SUBCORES
PER_W = TOTAL // NW
CHUNK = 1280
NCHUNK = PER_W // CHUNK

_mesh = plsc.VectorSubcoreMesh(core_axis_name="c", subcore_axis_name="s")


@functools.partial(
    pl.kernel,
    mesh=_mesh,
    out_type=jax.ShapeDtypeStruct((TOTAL, DIM), jnp.float32),
    scratch_types=[
        pltpu.VMEM((CHUNK,), jnp.int32),
        pltpu.VMEM((CHUNK, DIM), jnp.float32),
        pltpu.SemaphoreType.DMA,
    ],
    compiler_params=pltpu.CompilerParams(use_tc_tiling_on_sc=False),
)
def _embed(tokens_hbm, table_hbm, out_hbm, idx_v, rows_v, sem):
    wid = lax.axis_index("s") * NUM_CORES + lax.axis_index("c")
    base = wid * PER_W

    def body(c, carry):
        b0 = base + c * CHUNK
        pltpu.sync_copy(tokens_hbm.at[pl.ds(b0, CHUNK)], idx_v)
        pltpu.async_copy(table_hbm.at[idx_v], rows_v, sem).wait()
        pltpu.sync_copy(rows_v, out_hbm.at[pl.ds(b0, CHUNK)])
        return carry

    lax.fori_loop(0, NCHUNK, body, 0)


def kernel(tokens, table):
    flat = tokens.reshape(TOTAL).astype(jnp.int32)
    out = _embed(flat, table)
    return out.reshape(BATCH, HIST, DIM)


# speedup vs baseline: 1.4973x; 1.0188x over previous
import functools

import jax
import jax.numpy as jnp
from jax import lax
from jax.experimental import pallas as pl
from jax.experimental.pallas import tpu as pltpu
from jax.experimental.pallas import tpu_sc as plsc

DIM = 32
BATCH = 4096
HIST = 200
TOTAL = BATCH * HIST
NUM_CORES = 2
NUM_SUBCORES = 16
NW = NUM_CORES * NUM_SUBCORES
PER_W = TOTAL // NW
CHUNK = 1280
NCHUNK = PER_W // CHUNK
NBUF = 3

_mesh = plsc.VectorSubcoreMesh(core_axis_name="c", subcore_axis_name="s")


@functools.partial(
    pl.kernel,
    mesh=_mesh,
    out_type=jax.ShapeDtypeStruct((TOTAL, DIM), jnp.float32),
    scratch_types=(
        [pltpu.VMEM((CHUNK,), jnp.int32) for _ in range(NBUF)]
        + [pltpu.VMEM((CHUNK, DIM), jnp.float32) for _ in range(NBUF)]
        + [pltpu.SemaphoreType.DMA for _ in range(3 * NBUF)]
    ),
    compiler_params=pltpu.CompilerParams(use_tc_tiling_on_sc=False),
)
def _embed(tokens_hbm, table_hbm, out_hbm, *scratch):
    idx_v = scratch[:NBUF]
    rows_v = scratch[NBUF:2 * NBUF]
    idx_sem = scratch[2 * NBUF:3 * NBUF]
    g_sem = scratch[3 * NBUF:4 * NBUF]
    out_sem = scratch[4 * NBUF:5 * NBUF]
    wid = lax.axis_index("s") * NUM_CORES + lax.axis_index("c")
    base = wid * PER_W

    def idx_start(c, b):
        pltpu.async_copy(tokens_hbm.at[pl.ds(base + c * CHUNK, CHUNK)],
                         idx_v[b], idx_sem[b])

    def idx_wait(b):
        pltpu.make_async_copy(tokens_hbm.at[pl.ds(0, CHUNK)], idx_v[b],
                              idx_sem[b]).wait()

    def gather_start(b):
        pltpu.async_copy(table_hbm.at[idx_v[b]], rows_v[b], g_sem[b])

    def gather_wait(b):
        pltpu.make_async_copy(table_hbm.at[idx_v[b]], rows_v[b],
                              g_sem[b]).wait()

    def out_start(c, b):
        pltpu.async_copy(rows_v[b],
                         out_hbm.at[pl.ds(base + c * CHUNK, CHUNK)],
                         out_sem[b])

    def out_wait(b):
        pltpu.make_async_copy(rows_v[b],
                              out_hbm.at[pl.ds(0, CHUNK)],
                              out_sem[b]).wait()

    for b in range(NBUF):
        idx_start(b, b)

    for c in range(NCHUNK):
        b = c % NBUF
        idx_wait(b)
        if c >= NBUF:
            out_wait(b)
        gather_start(b)
        if c >= 1:
            bp = (c - 1) % NBUF
            gather_wait(bp)
            out_start(c - 1, bp)
            if c - 1 + NBUF < NCHUNK:
                idx_start(c - 1 + NBUF, bp)

    bl = (NCHUNK - 1) % NBUF
    gather_wait(bl)
    out_start(NCHUNK - 1, bl)
    for b in range(NBUF):
        out_wait(b)


def kernel(tokens, table):
    flat = tokens.reshape(TOTAL).astype(jnp.int32)
    out = _embed(flat, table)
    return out.reshape(BATCH, HIST, DIM)
